# Initial kernel scaffold; baseline (speedup 1.0000x reference)
import jax
import jax.numpy as jnp
from jax.experimental import pallas as pl


def kernel(x, w1m, b1c, w2m, b2c, w3p, b3r, w4t, b4r):
    raise NotImplementedError("write your pallas kernel here")


# speedup vs baseline: 3.8175x; 3.8175x over previous
import jax
import jax.numpy as jnp
from jax.experimental import pallas as pl
from jax.experimental.pallas import tpu as pltpu

_H = _W = 28
_KH = _KW = 3
_C1, _C2 = 20, 10
_FCH, _FCO = 100, 10

_M2E = (_H - 4) * _W
_M1E = _M2E + 2 * _W + 2
_KFC = _C2 * _M2E
_B = 128

_CH = 32
_N1 = 23
_N2 = 21
_M1P = _N1 * _CH
_L0P = _M1P + 2 * _W + 2
_OFFS = tuple(dy * _W + dx for dy in range(_KH) for dx in range(_KW))


def _fused_kernel(x_ref, w1_ref, b1_ref, w2_ref, b2_ref, w3_ref, b3_ref,
                  w4_ref, b4_ref, o_ref, y1_scr, y2_scr):
    def c1_body(c, carry):
        j0 = c * _CH
        p1 = jnp.stack([x_ref[pl.ds(j0 + off, _CH), :] for off in _OFFS],
                       axis=0)
        a1 = jax.lax.dot_general(w1_ref[...], p1, (((1,), (0,)), ((), ())),
                                 precision=jax.lax.Precision.HIGHEST,
                                 preferred_element_type=jnp.float32)
        y1_scr[:, pl.ds(j0 * _B, _CH * _B)] = jnp.tanh(
            a1.reshape(_C1, _CH * _B) + b1_ref[...])
        return carry

    jax.lax.fori_loop(0, _N1, c1_body, 0, unroll=False)

    def c2_body(c, carry):
        j0 = c * _CH
        acc = jnp.zeros((_C2, _CH * _B), jnp.float32)
        for tap, off in enumerate(_OFFS):
            acc = acc + jnp.dot(
                w2_ref[tap], y1_scr[:, pl.ds((j0 + off) * _B, _CH * _B)],
                preferred_element_type=jnp.float32)
        y2 = jnp.tanh(acc + b2_ref[...])
        for c2 in range(_C2):
            y2_scr[pl.ds(c2 * _M2E + j0, _CH), :] = y2[c2].reshape(_CH, _B)
        return carry

    jax.lax.fori_loop(0, _N2, c2_body, 0, unroll=False)

    h = jnp.tanh(
        jnp.dot(w3_ref[...], y2_scr[...], preferred_element_type=jnp.float32)
        + b3_ref[...])
    o_ref[...] = (
        jnp.dot(w4_ref[...], h, preferred_element_type=jnp.float32)
        + b4_ref[...])


def kernel(x, w1m, b1c, w2m, b2c, w3p, b3r, w4t, b4r):
    n = x.shape[0]
    nb = pl.cdiv(n, _B)
    n_pad = nb * _B

    xf = x.astype(jnp.float32).reshape(n, _H * _W)
    if n_pad != n:
        xf = jnp.pad(xf, ((0, n_pad - n), (0, 0)))
    xt = jnp.pad(xf, ((0, 0), (0, _L0P - _H * _W))).T
    w1r = w1m[:, :, 0].T
    w3t = w3p.T
    w4tt = w4t.T
    b3c = b3r.reshape(_FCH, 1)
    b4c = b4r.reshape(_FCO, 1)

    flops = n_pad * (2 * 9 * _C1 * _M1E + 2 * 9 * _C2 * _C1 * _M2E
                     + 2 * (_KFC * _FCH + _FCH * _FCO))
    trans = n_pad * (_C1 * _M1E + _C2 * _M2E + _FCH)
    byts = 4 * (n_pad * (_L0P + _FCO) + w1r.size + b1c.size + w2m.size
                + b2c.size + w3t.size + b3c.size + w4tt.size + b4c.size)

    out = pl.pallas_call(
        _fused_kernel,
        out_shape=jax.ShapeDtypeStruct((_FCO, n_pad), jnp.float32),
        grid=(nb,),
        in_specs=[
            pl.BlockSpec((_L0P, _B), lambda i: (0, i)),
            pl.BlockSpec((_C1, _KH * _KW), lambda i: (0, 0)),
            pl.BlockSpec((_C1, 1), lambda i: (0, 0)),
            pl.BlockSpec((_KH * _KW, _C2, _C1), lambda i: (0, 0, 0)),
            pl.BlockSpec((_C2, 1), lambda i: (0, 0)),
            pl.BlockSpec((_FCH, _KFC), lambda i: (0, 0)),
            pl.BlockSpec((_FCH, 1), lambda i: (0, 0)),
            pl.BlockSpec((_FCO, _FCH), lambda i: (0, 0)),
            pl.BlockSpec((_FCO, 1), lambda i: (0, 0)),
        ],
        out_specs=pl.BlockSpec((_FCO, _B), lambda i: (0, i)),
        scratch_shapes=[
            pltpu.VMEM((_C1, _M1P * _B), jnp.float32),
            pltpu.VMEM((_KFC, _B), jnp.float32),
        ],
        compiler_params=pltpu.CompilerParams(
            dimension_semantics=("parallel",),
            vmem_limit_bytes=100 * 1024 * 1024),
        cost_estimate=pl.CostEstimate(flops=flops, transcendentals=trans,
                                      bytes_accessed=byts),
    )(xt, w1r, b1c, w2m, b2c, w3t, b3c, w4tt, b4c)
    return out.T[:n]


# speedup vs baseline: 5.3206x; 1.3937x over previous
import jax
import jax.numpy as jnp
from jax.experimental import pallas as pl
from jax.experimental.pallas import tpu as pltpu

_H = _W = 28
_KH = _KW = 3
_C1, _C2 = 20, 10
_FCH, _FCO = 100, 10

_M2E = (_H - 4) * _W
_M1E = _M2E + 2 * _W + 2
_KFC = _C2 * _M2E
_B = 128

_CH = 96
_N1 = 8
_N2 = 7
_M1P = _N1 * _CH
_L0P = _M1P + 2 * _W + 2
_OFFS = tuple(dy * _W + dx for dy in range(_KH) for dx in range(_KW))


def _fused_kernel(x_ref, w1_ref, b1_ref, w2_ref, b2_ref, w3_ref, b3_ref,
                  w4_ref, b4_ref, o_ref, y1_scr, y2_scr):
    def c1_body(c, carry):
        j0 = c * _CH
        p1 = jnp.stack([x_ref[pl.ds(j0 + off, _CH), :] for off in _OFFS],
                       axis=0)
        a1 = jax.lax.dot_general(w1_ref[...], p1, (((1,), (0,)), ((), ())),
                                 precision=jax.lax.Precision.HIGHEST,
                                 preferred_element_type=jnp.float32)
        y1_scr[:, pl.ds(j0 * _B, _CH * _B)] = jnp.tanh(
            a1.reshape(_C1, _CH * _B) + b1_ref[...])
        return carry

    for c in range(_N1):
        c1_body(c, 0)

    def c2_body(c, carry):
        j0 = c * _CH
        acc = jnp.zeros((_C2, _CH * _B), jnp.float32)
        for tap, off in enumerate(_OFFS):
            acc = acc + jnp.dot(
                w2_ref[tap], y1_scr[:, pl.ds((j0 + off) * _B, _CH * _B)],
                preferred_element_type=jnp.float32)
        y2 = jnp.tanh(acc + b2_ref[...])
        for c2 in range(_C2):
            y2_scr[pl.ds(c2 * _M2E + j0, _CH), :] = y2[c2].reshape(_CH, _B)
        return carry

    for c in range(_N2):
        c2_body(c, 0)

    h = jnp.tanh(
        jnp.dot(w3_ref[...], y2_scr[...], preferred_element_type=jnp.float32)
        + b3_ref[...])
    o_ref[...] = (
        jnp.dot(w4_ref[...], h, preferred_element_type=jnp.float32)
        + b4_ref[...])


def kernel(x, w1m, b1c, w2m, b2c, w3p, b3r, w4t, b4r):
    n = x.shape[0]
    nb = pl.cdiv(n, _B)
    n_pad = nb * _B

    xf = x.astype(jnp.float32).reshape(n, _H * _W)
    if n_pad != n:
        xf = jnp.pad(xf, ((0, n_pad - n), (0, 0)))
    xt = jnp.pad(xf, ((0, 0), (0, _L0P - _H * _W))).T
    w1r = w1m[:, :, 0].T
    w3t = w3p.T
    w4tt = w4t.T
    b3c = b3r.reshape(_FCH, 1)
    b4c = b4r.reshape(_FCO, 1)

    flops = n_pad * (2 * 9 * _C1 * _M1E + 2 * 9 * _C2 * _C1 * _M2E
                     + 2 * (_KFC * _FCH + _FCH * _FCO))
    trans = n_pad * (_C1 * _M1E + _C2 * _M2E + _FCH)
    byts = 4 * (n_pad * (_L0P + _FCO) + w1r.size + b1c.size + w2m.size
                + b2c.size + w3t.size + b3c.size + w4tt.size + b4c.size)

    out = pl.pallas_call(
        _fused_kernel,
        out_shape=jax.ShapeDtypeStruct((_FCO, n_pad), jnp.float32),
        grid=(nb,),
        in_specs=[
            pl.BlockSpec((_L0P, _B), lambda i: (0, i)),
            pl.BlockSpec((_C1, _KH * _KW), lambda i: (0, 0)),
            pl.BlockSpec((_C1, 1), lambda i: (0, 0)),
            pl.BlockSpec((_KH * _KW, _C2, _C1), lambda i: (0, 0, 0)),
            pl.BlockSpec((_C2, 1), lambda i: (0, 0)),
            pl.BlockSpec((_FCH, _KFC), lambda i: (0, 0)),
            pl.BlockSpec((_FCH, 1), lambda i: (0, 0)),
            pl.BlockSpec((_FCO, _FCH), lambda i: (0, 0)),
            pl.BlockSpec((_FCO, 1), lambda i: (0, 0)),
        ],
        out_specs=pl.BlockSpec((_FCO, _B), lambda i: (0, i)),
        scratch_shapes=[
            pltpu.VMEM((_C1, _M1P * _B), jnp.float32),
            pltpu.VMEM((_KFC, _B), jnp.float32),
        ],
        compiler_params=pltpu.CompilerParams(
            dimension_semantics=("parallel",),
            vmem_limit_bytes=100 * 1024 * 1024),
        cost_estimate=pl.CostEstimate(flops=flops, transcendentals=trans,
                                      bytes_accessed=byts),
    )(xt, w1r, b1c, w2m, b2c, w3t, b3c, w4tt, b4c)
    return out.T[:n]


# speedup vs baseline: 5.3504x; 1.0056x over previous
import jax
import jax.numpy as jnp
from jax.experimental import pallas as pl
from jax.experimental.pallas import tpu as pltpu

_H = _W = 28
_KH = _KW = 3
_C1, _C2 = 20, 10
_FCH, _FCO = 100, 10

_M2E = (_H - 4) * _W
_M1E = _M2E + 2 * _W + 2
_KFC = _C2 * _M2E
_B = 128

_CH = 96
_N1 = 8
_N2 = 7
_M1P = _N1 * _CH
_L0P = _M1P + 2 * _W + 2
_OFFS = tuple(dy * _W + dx for dy in range(_KH) for dx in range(_KW))
_HW = _H * _W


def _fused_kernel(x_ref, w1_ref, b1_ref, w2_ref, b2_ref, w3_ref, b3_ref,
                  w4_ref, b4_ref, o_ref, xt_scr, y1_scr, y2_scr):
    xt_scr[pl.ds(0, _HW), :] = x_ref[...].T
    xt_scr[pl.ds(_HW, _L0P - _HW), :] = jnp.zeros((_L0P - _HW, _B),
                                                  jnp.float32)
    def c1_body(c, carry):
        j0 = c * _CH
        p1 = jnp.stack([xt_scr[pl.ds(j0 + off, _CH), :] for off in _OFFS],
                       axis=0)
        a1 = jax.lax.dot_general(w1_ref[...], p1, (((1,), (0,)), ((), ())),
                                 precision=jax.lax.Precision.HIGHEST,
                                 preferred_element_type=jnp.float32)
        y1_scr[:, pl.ds(j0 * _B, _CH * _B)] = jnp.tanh(
            a1.reshape(_C1, _CH * _B) + b1_ref[...])
        return carry

    for c in range(_N1):
        c1_body(c, 0)

    def c2_body(c, carry):
        j0 = c * _CH
        acc = jnp.zeros((_C2, _CH * _B), jnp.float32)
        for tap, off in enumerate(_OFFS):
            acc = acc + jnp.dot(
                w2_ref[tap], y1_scr[:, pl.ds((j0 + off) * _B, _CH * _B)],
                preferred_element_type=jnp.float32)
        y2 = jnp.tanh(acc + b2_ref[...])
        for c2 in range(_C2):
            y2_scr[pl.ds(c2 * _M2E + j0, _CH), :] = y2[c2].reshape(_CH, _B)
        return carry

    for c in range(_N2):
        c2_body(c, 0)

    h = jnp.tanh(
        jnp.dot(w3_ref[...], y2_scr[...], preferred_element_type=jnp.float32)
        + b3_ref[...])
    o_ref[...] = (
        jnp.dot(w4_ref[...], h, preferred_element_type=jnp.float32)
        + b4_ref[...])


def kernel(x, w1m, b1c, w2m, b2c, w3p, b3r, w4t, b4r):
    n = x.shape[0]
    nb = pl.cdiv(n, _B)
    n_pad = nb * _B

    xf = x.astype(jnp.float32).reshape(n, _H * _W)
    if n_pad != n:
        xf = jnp.pad(xf, ((0, n_pad - n), (0, 0)))
    w1r = w1m[:, :, 0].T
    w3t = w3p.T
    w4tt = w4t.T
    b3c = b3r.reshape(_FCH, 1)
    b4c = b4r.reshape(_FCO, 1)

    flops = n_pad * (2 * 9 * _C1 * _M1E + 2 * 9 * _C2 * _C1 * _M2E
                     + 2 * (_KFC * _FCH + _FCH * _FCO))
    trans = n_pad * (_C1 * _M1E + _C2 * _M2E + _FCH)
    byts = 4 * (n_pad * (_L0P + _FCO) + w1r.size + b1c.size + w2m.size
                + b2c.size + w3t.size + b3c.size + w4tt.size + b4c.size)

    out = pl.pallas_call(
        _fused_kernel,
        out_shape=jax.ShapeDtypeStruct((_FCO, n_pad), jnp.float32),
        grid=(nb,),
        in_specs=[
            pl.BlockSpec((_B, _HW), lambda i: (i, 0)),
            pl.BlockSpec((_C1, _KH * _KW), lambda i: (0, 0)),
            pl.BlockSpec((_C1, 1), lambda i: (0, 0)),
            pl.BlockSpec((_KH * _KW, _C2, _C1), lambda i: (0, 0, 0)),
            pl.BlockSpec((_C2, 1), lambda i: (0, 0)),
            pl.BlockSpec((_FCH, _KFC), lambda i: (0, 0)),
            pl.BlockSpec((_FCH, 1), lambda i: (0, 0)),
            pl.BlockSpec((_FCO, _FCH), lambda i: (0, 0)),
            pl.BlockSpec((_FCO, 1), lambda i: (0, 0)),
        ],
        out_specs=pl.BlockSpec((_FCO, _B), lambda i: (0, i)),
        scratch_shapes=[
            pltpu.VMEM((_L0P, _B), jnp.float32),
            pltpu.VMEM((_C1, _M1P * _B), jnp.float32),
            pltpu.VMEM((_KFC, _B), jnp.float32),
        ],
        compiler_params=pltpu.CompilerParams(
            dimension_semantics=("parallel",),
            vmem_limit_bytes=100 * 1024 * 1024),
        cost_estimate=pl.CostEstimate(flops=flops, transcendentals=trans,
                                      bytes_accessed=byts),
    )(xf, w1r, b1c, w2m, b2c, w3t, b3c, w4tt, b4c)
    return out.T[:n]
